# baseline (speedup 1.0000x reference)
import functools

import jax
import jax.numpy as jnp
from jax.experimental import pallas as pl
from jax.experimental.pallas import tpu as pltpu

_K = 20
_N = 2048
_TN = 256
_JC = 256


def _lrelu(v):
    return jnp.where(v > 0, v, 0.2 * v)


def _dot_t(a, b):
    return jax.lax.dot_general(a, b, (((1,), (1,)), ((), ())),
                               preferred_element_type=jnp.float32)


def _main_kernel(x_tile_ref, xT_ref, x_full_ref, W0T_ref, b0_ref,
                 W1_ref, b1_ref, W2_ref, b2_ref, W3_ref, b3_ref, out_ref):
    t = pl.program_id(1)

    x_tile = x_tile_ref[0]
    xT = xT_ref[0]
    x_full = x_full_ref[0]
    W0T = W0T_ref[...]
    A_T = W0T[:3, :]
    D_T = W0T[3:, :] - A_T

    xx_full = jnp.sum(xT * xT, axis=0)[None, :]
    xx_tile = jnp.sum(x_tile * x_tile, axis=1, keepdims=True)
    inner = jnp.dot(x_tile, xT, preferred_element_type=jnp.float32)
    dist = xx_tile - 2.0 * inner + xx_full

    u_full = jnp.dot(x_full, A_T, preferred_element_type=jnp.float32)
    c_i = jnp.dot(x_tile, D_T, preferred_element_type=jnp.float32) + b0_ref[...]
    u_hi = u_full.astype(jnp.bfloat16)
    u_lo = (u_full - u_hi.astype(jnp.float32)).astype(jnp.bfloat16)
    u_cat = jnp.concatenate([u_hi, u_lo], axis=1)


    iota = jax.lax.broadcasted_iota(jnp.int32, (_TN, _N), 1).astype(jnp.float32)
    big_f = jnp.float32(1e9)
    inf_f = jnp.float32(3e38)
    neg_inf = jnp.float32(-3e38)
    m = jnp.full((_TN, 64), neg_inf, jnp.float32)

    d = dist
    for _ in range(_K):
        curmin = jnp.min(d, axis=1, keepdims=True)
        cand = jnp.where(d == curmin, iota, big_f)
        idxmin = jnp.min(cand, axis=1, keepdims=True)
        onehot = cand == idxmin
        g2 = jnp.dot(onehot.astype(jnp.bfloat16), u_cat,
                     preferred_element_type=jnp.float32)
        m = jnp.maximum(m, g2[:, :64] + g2[:, 64:])
        d = jnp.where(onehot, inf_f, d)

    h = _lrelu(m + c_i)
    h = _lrelu(_dot_t(h, W1_ref[...]) + b1_ref[...])
    h = _lrelu(_dot_t(h, W2_ref[...]) + b2_ref[...])
    h = _lrelu(_dot_t(h, W3_ref[...]) + b3_ref[...])

    pmax = jnp.max(h, axis=0, keepdims=True)[None]

    @pl.when(t == 0)
    def _():
        out_ref[...] = pmax

    @pl.when(t != 0)
    def _():
        out_ref[...] = jnp.maximum(out_ref[...], pmax)


def _head_kernel(h_ref, W4_ref, b4_ref, W5_ref, b5_ref, out_ref):
    h = _lrelu(_dot_t(h_ref[...], W4_ref[...]) + b4_ref[...])
    out_ref[...] = _dot_t(h, W5_ref[...]) + b5_ref[...]


@jax.jit
def kernel(x, W0, b0, W1, b1, W2, b2, W3, b3, W4, b4, W5, b5):
    B = x.shape[0]
    xT = jnp.transpose(x, (0, 2, 1))
    W0T = W0.T
    b0r, b1r, b2r, b3r = (b.reshape(1, -1) for b in (b0, b1, b2, b3))

    n_tiles = _N // _TN
    hmax = pl.pallas_call(
        _main_kernel,
        grid=(B, n_tiles),
        in_specs=[
            pl.BlockSpec((1, _TN, 3), lambda b, t: (b, t, 0)),
            pl.BlockSpec((1, 3, _N), lambda b, t: (b, 0, 0)),
            pl.BlockSpec((1, _N, 3), lambda b, t: (b, 0, 0)),
            pl.BlockSpec((6, 64), lambda b, t: (0, 0)),
            pl.BlockSpec((1, 64), lambda b, t: (0, 0)),
            pl.BlockSpec((64, 64), lambda b, t: (0, 0)),
            pl.BlockSpec((1, 64), lambda b, t: (0, 0)),
            pl.BlockSpec((128, 64), lambda b, t: (0, 0)),
            pl.BlockSpec((1, 128), lambda b, t: (0, 0)),
            pl.BlockSpec((128, 128), lambda b, t: (0, 0)),
            pl.BlockSpec((1, 128), lambda b, t: (0, 0)),
        ],
        out_specs=pl.BlockSpec((1, 1, 128), lambda b, t: (b, 0, 0)),
        out_shape=jax.ShapeDtypeStruct((B, 1, 128), jnp.float32),
        compiler_params=pltpu.CompilerParams(
            dimension_semantics=("arbitrary", "arbitrary")),
    )(x, xT, x, W0T, b0r, W1, b1r, W2, b2r, W3, b3r)

    out = pl.pallas_call(
        _head_kernel,
        out_shape=jax.ShapeDtypeStruct((B, 1024), jnp.float32),
    )(hmax.reshape(B, 128), W4, b4.reshape(1, -1), W5, b5.reshape(1, -1))
    return out


# speedup vs baseline: 1.0355x; 1.0355x over previous
import functools

import jax
import jax.numpy as jnp
from jax.experimental import pallas as pl
from jax.experimental.pallas import tpu as pltpu

_K = 20
_N = 2048
_TN = 256
_JC = 256


def _lrelu(v):
    return jnp.where(v > 0, v, 0.2 * v)


def _dot_t(a, b):
    return jax.lax.dot_general(a, b, (((1,), (1,)), ((), ())),
                               preferred_element_type=jnp.float32)


def _main_kernel(x_tile_ref, xT_ref, x_full_ref, W0T_ref, b0_ref,
                 W1_ref, b1_ref, W2_ref, b2_ref, W3_ref, b3_ref, out_ref):
    t = pl.program_id(1)

    x_tile = x_tile_ref[0]
    xT = xT_ref[0]
    x_full = x_full_ref[0]
    W0T = W0T_ref[...]
    A_T = W0T[:3, :]
    D_T = W0T[3:, :] - A_T

    xx_full = jnp.sum(xT * xT, axis=0)[None, :]
    xx_tile = jnp.sum(x_tile * x_tile, axis=1, keepdims=True)
    inner = jnp.dot(x_tile, xT, preferred_element_type=jnp.float32)
    dist = xx_tile - 2.0 * inner + xx_full

    u_full = jnp.dot(x_full, A_T, preferred_element_type=jnp.float32)
    c_i = jnp.dot(x_tile, D_T, preferred_element_type=jnp.float32) + b0_ref[...]


    _TH = _TN // 2
    iota = jax.lax.broadcasted_iota(jnp.int32, (_TH, _N), 1).astype(jnp.float32)
    big_f = jnp.float32(1e9)
    inf_f = jnp.float32(3e38)
    neg_inf = jnp.float32(-3e38)

    halves = [[dist[:_TH], jnp.full((_TH, 64), neg_inf, jnp.float32)],
              [dist[_TH:], jnp.full((_TH, 64), neg_inf, jnp.float32)]]
    for _ in range(_K):
        for hv in halves:
            d, m = hv
            curmin = jnp.min(d, axis=1, keepdims=True)
            cand = jnp.where(d == curmin, iota, big_f)
            idxmin = jnp.min(cand, axis=1, keepdims=True)
            onehot = cand == idxmin
            g = jnp.dot(onehot.astype(jnp.float32), u_full,
                        preferred_element_type=jnp.float32)
            hv[0] = jnp.where(onehot, inf_f, d)
            hv[1] = jnp.maximum(m, g)
    m = jnp.concatenate([halves[0][1], halves[1][1]], axis=0)

    h = _lrelu(m + c_i)
    h = _lrelu(_dot_t(h, W1_ref[...]) + b1_ref[...])
    h = _lrelu(_dot_t(h, W2_ref[...]) + b2_ref[...])
    h = _lrelu(_dot_t(h, W3_ref[...]) + b3_ref[...])

    pmax = jnp.max(h, axis=0, keepdims=True)[None]

    @pl.when(t == 0)
    def _():
        out_ref[...] = pmax

    @pl.when(t != 0)
    def _():
        out_ref[...] = jnp.maximum(out_ref[...], pmax)


def _head_kernel(h_ref, W4_ref, b4_ref, W5_ref, b5_ref, out_ref):
    h = _lrelu(_dot_t(h_ref[...], W4_ref[...]) + b4_ref[...])
    out_ref[...] = _dot_t(h, W5_ref[...]) + b5_ref[...]


@jax.jit
def kernel(x, W0, b0, W1, b1, W2, b2, W3, b3, W4, b4, W5, b5):
    B = x.shape[0]
    xT = jnp.transpose(x, (0, 2, 1))
    W0T = W0.T
    b0r, b1r, b2r, b3r = (b.reshape(1, -1) for b in (b0, b1, b2, b3))

    n_tiles = _N // _TN
    hmax = pl.pallas_call(
        _main_kernel,
        grid=(B, n_tiles),
        in_specs=[
            pl.BlockSpec((1, _TN, 3), lambda b, t: (b, t, 0)),
            pl.BlockSpec((1, 3, _N), lambda b, t: (b, 0, 0)),
            pl.BlockSpec((1, _N, 3), lambda b, t: (b, 0, 0)),
            pl.BlockSpec((6, 64), lambda b, t: (0, 0)),
            pl.BlockSpec((1, 64), lambda b, t: (0, 0)),
            pl.BlockSpec((64, 64), lambda b, t: (0, 0)),
            pl.BlockSpec((1, 64), lambda b, t: (0, 0)),
            pl.BlockSpec((128, 64), lambda b, t: (0, 0)),
            pl.BlockSpec((1, 128), lambda b, t: (0, 0)),
            pl.BlockSpec((128, 128), lambda b, t: (0, 0)),
            pl.BlockSpec((1, 128), lambda b, t: (0, 0)),
        ],
        out_specs=pl.BlockSpec((1, 1, 128), lambda b, t: (b, 0, 0)),
        out_shape=jax.ShapeDtypeStruct((B, 1, 128), jnp.float32),
        compiler_params=pltpu.CompilerParams(
            dimension_semantics=("arbitrary", "arbitrary")),
    )(x, xT, x, W0T, b0r, W1, b1r, W2, b2r, W3, b3r)

    out = pl.pallas_call(
        _head_kernel,
        out_shape=jax.ShapeDtypeStruct((B, 1024), jnp.float32),
    )(hmax.reshape(B, 128), W4, b4.reshape(1, -1), W5, b5.reshape(1, -1))
    return out


# speedup vs baseline: 1.3606x; 1.3140x over previous
import functools

import jax
import jax.numpy as jnp
from jax.experimental import pallas as pl
from jax.experimental.pallas import tpu as pltpu

_K = 20
_N = 2048
_TN = 256
_JC = 256


def _lrelu(v):
    return jnp.where(v > 0, v, 0.2 * v)


def _dot_t(a, b):
    return jax.lax.dot_general(a, b, (((1,), (1,)), ((), ())),
                               preferred_element_type=jnp.float32)


def _main_kernel(x_tile_ref, xT_ref, x_full_ref, W0T_ref, b0_ref,
                 W1_ref, b1_ref, W2_ref, b2_ref, W3_ref, b3_ref, out_ref):
    t = pl.program_id(1)

    x_tile = x_tile_ref[0]
    xT = xT_ref[0]
    x_full = x_full_ref[0]
    W0T = W0T_ref[...]
    A_T = W0T[:3, :]
    D_T = W0T[3:, :] - A_T

    xx_full = jnp.sum(xT * xT, axis=0)[None, :]
    xx_tile = jnp.sum(x_tile * x_tile, axis=1, keepdims=True)
    inner = jnp.dot(x_tile, xT, preferred_element_type=jnp.float32)
    dist = xx_tile - 2.0 * inner + xx_full

    u_full = jnp.dot(x_full, A_T, preferred_element_type=jnp.float32)
    c_i = jnp.dot(x_tile, D_T, preferred_element_type=jnp.float32) + b0_ref[...]


    iota = jax.lax.broadcasted_iota(jnp.int32, (_TN, _N), 1).astype(jnp.float32)
    big_f = jnp.float32(1e9)
    inf_f = jnp.float32(3e38)

    row_id = (jax.lax.broadcasted_iota(jnp.int32, (_TN, 1), 0)
              + t * _TN).astype(jnp.float32)
    m = jnp.dot(x_tile, A_T, preferred_element_type=jnp.float32)
    d = jnp.where(iota == row_id, inf_f, dist)

    for _ in range(_K - 1):
        curmin = jnp.min(d, axis=1, keepdims=True)
        cand = jnp.where(d == curmin, iota, big_f)
        idxmin = jnp.min(cand, axis=1, keepdims=True)
        onehot = cand == idxmin
        g = jnp.dot(onehot.astype(jnp.float32), u_full,
                    preferred_element_type=jnp.float32)
        m = jnp.maximum(m, g)
        d = jnp.where(onehot, inf_f, d)

    h = _lrelu(m + c_i)
    h = _lrelu(_dot_t(h, W1_ref[...]) + b1_ref[...])
    h = _lrelu(_dot_t(h, W2_ref[...]) + b2_ref[...])
    h = _lrelu(_dot_t(h, W3_ref[...]) + b3_ref[...])

    pmax = jnp.max(h, axis=0, keepdims=True)[None]

    @pl.when(t == 0)
    def _():
        out_ref[...] = pmax

    @pl.when(t != 0)
    def _():
        out_ref[...] = jnp.maximum(out_ref[...], pmax)


def _head_kernel(h_ref, W4_ref, b4_ref, W5_ref, b5_ref, out_ref):
    h = _lrelu(_dot_t(h_ref[...], W4_ref[...]) + b4_ref[...])
    out_ref[...] = _dot_t(h, W5_ref[...]) + b5_ref[...]


@jax.jit
def kernel(x, W0, b0, W1, b1, W2, b2, W3, b3, W4, b4, W5, b5):
    B = x.shape[0]
    xT = jnp.transpose(x, (0, 2, 1))
    W0T = W0.T
    b0r, b1r, b2r, b3r = (b.reshape(1, -1) for b in (b0, b1, b2, b3))

    n_tiles = _N // _TN
    hmax = pl.pallas_call(
        _main_kernel,
        grid=(B, n_tiles),
        in_specs=[
            pl.BlockSpec((1, _TN, 3), lambda b, t: (b, t, 0)),
            pl.BlockSpec((1, 3, _N), lambda b, t: (b, 0, 0)),
            pl.BlockSpec((1, _N, 3), lambda b, t: (b, 0, 0)),
            pl.BlockSpec((6, 64), lambda b, t: (0, 0)),
            pl.BlockSpec((1, 64), lambda b, t: (0, 0)),
            pl.BlockSpec((64, 64), lambda b, t: (0, 0)),
            pl.BlockSpec((1, 64), lambda b, t: (0, 0)),
            pl.BlockSpec((128, 64), lambda b, t: (0, 0)),
            pl.BlockSpec((1, 128), lambda b, t: (0, 0)),
            pl.BlockSpec((128, 128), lambda b, t: (0, 0)),
            pl.BlockSpec((1, 128), lambda b, t: (0, 0)),
        ],
        out_specs=pl.BlockSpec((1, 1, 128), lambda b, t: (b, 0, 0)),
        out_shape=jax.ShapeDtypeStruct((B, 1, 128), jnp.float32),
        compiler_params=pltpu.CompilerParams(
            dimension_semantics=("arbitrary", "arbitrary")),
    )(x, xT, x, W0T, b0r, W1, b1r, W2, b2r, W3, b3r)

    out = pl.pallas_call(
        _head_kernel,
        out_shape=jax.ShapeDtypeStruct((B, 1024), jnp.float32),
    )(hmax.reshape(B, 128), W4, b4.reshape(1, -1), W5, b5.reshape(1, -1))
    return out
